# baseline (speedup 1.0000x reference)
import functools

import jax
import jax.numpy as jnp
from jax import lax
from jax.experimental import pallas as pl
from jax.experimental.pallas import tpu as pltpu
from jax.experimental.pallas import tpu_sc as plsc

VOCAB = 100000
EMBED = 128
BATCH = 1024
SEQ = 200

_info = plsc.get_sparse_core_info()
_NC, _NS = _info.num_cores, _info.num_subcores
_NW = _NC * _NS
_K = 4
_NB = BATCH // _K
_BPW = _NB // _NW
_HALF = SEQ // 2
_NBUF = 4


@functools.partial(
    pl.kernel,
    mesh=plsc.VectorSubcoreMesh(core_axis_name="c", subcore_axis_name="s"),
    out_type=jax.ShapeDtypeStruct((_NB, SEQ, EMBED), jnp.float32),
    scratch_types=(
        [pltpu.VMEM((2 * _BPW, _HALF), jnp.int32)]
        + [pltpu.VMEM((SEQ, EMBED), jnp.float32)] * _NBUF
        + [pltpu.SemaphoreType.DMA] * (2 * _NBUF)
    ),
)
def _gather_kernel(idx_hbm, table_hbm, out_hbm, idx_v, *bufs_and_sems):
    rows_bufs = bufs_and_sems[:_NBUF]
    gsems = bufs_and_sems[_NBUF:2 * _NBUF]
    osems = bufs_and_sems[2 * _NBUF:]
    bufs = tuple(zip(rows_bufs, gsems, osems))
    wid = lax.axis_index("s") * _NC + lax.axis_index("c")
    bbase = wid * _BPW
    pltpu.sync_copy(idx_hbm.at[wid], idx_v)

    def start_batch(i, rows, gsem):
        pltpu.async_copy(table_hbm.at[idx_v.at[2 * i]],
                         rows.at[pl.ds(0, _HALF)], gsem)
        pltpu.async_copy(table_hbm.at[idx_v.at[2 * i + 1]],
                         rows.at[pl.ds(_HALF, _HALF)], gsem)

    def wait_batch(i, rows, gsem):
        pltpu.make_async_copy(table_hbm.at[idx_v.at[2 * i]],
                              rows.at[pl.ds(0, _HALF)], gsem).wait()
        pltpu.make_async_copy(table_hbm.at[idx_v.at[2 * i + 1]],
                              rows.at[pl.ds(_HALF, _HALF)], gsem).wait()

    for b in range(_NBUF):
        start_batch(b, bufs[b][0], bufs[b][1])

    def body(g, _):
        for b in range(_NBUF):
            i = g * _NBUF + b
            rows, gsem, osem = bufs[b]
            wait_batch(i, rows, gsem)
            pltpu.async_copy(rows, out_hbm.at[bbase + i], osem)

            @pl.when(i + _NBUF < _BPW)
            def _():
                pltpu.make_async_copy(rows, out_hbm.at[bbase + i], osem).wait()
                start_batch(i + _NBUF, rows, gsem)

        return 0

    lax.fori_loop(0, _BPW // _NBUF, body, 0)

    for b in range(_NBUF):
        i = _BPW - _NBUF + b
        rows, _, osem = bufs[b]
        pltpu.make_async_copy(rows, out_hbm.at[bbase + i], osem).wait()


def kernel(inputs, embedding_table):
    idx = jnp.reshape(inputs.astype(jnp.int32), (_K, _NW, 2 * _BPW, _HALF))
    out = jnp.zeros((BATCH, SEQ, EMBED), jnp.float32)
    for k in range(_K):
        part = _gather_kernel(idx[k], embedding_table)
        out = lax.dynamic_update_slice(out, part, (k * _NB, 0, 0))
    return (out, embedding_table)


# speedup vs baseline: 1.0407x; 1.0407x over previous
import functools

import jax
import jax.numpy as jnp
from jax import lax
from jax.experimental import pallas as pl
from jax.experimental.pallas import tpu as pltpu
from jax.experimental.pallas import tpu_sc as plsc

VOCAB = 100000
EMBED = 128
BATCH = 1024
SEQ = 200

_info = plsc.get_sparse_core_info()
_NC, _NS = _info.num_cores, _info.num_subcores
_NW = _NC * _NS
_K = 4
_NB = BATCH // _K
_BPW = _NB // _NW
_HALF = SEQ // 2
_NBUF = 4


@functools.partial(
    pl.kernel,
    mesh=plsc.VectorSubcoreMesh(core_axis_name="c", subcore_axis_name="s"),
    out_type=jax.ShapeDtypeStruct((_NB, SEQ, EMBED), jnp.float32),
    scratch_types=(
        [pltpu.VMEM((2 * _BPW, _HALF), jnp.int32)]
        + [pltpu.VMEM((SEQ, EMBED), jnp.float32)] * _NBUF
        + [pltpu.SemaphoreType.DMA] * (2 * _NBUF)
    ),
)
def _gather_kernel(idx_hbm, table_hbm, out_hbm, idx_v, *bufs_and_sems):
    rows_bufs = bufs_and_sems[:_NBUF]
    gsems = bufs_and_sems[_NBUF:2 * _NBUF]
    osems = bufs_and_sems[2 * _NBUF:]
    bufs = tuple(zip(rows_bufs, gsems, osems))
    wid = lax.axis_index("s") * _NC + lax.axis_index("c")
    bbase = wid * _BPW
    pltpu.sync_copy(idx_hbm.at[wid], idx_v)

    def start_batch(i, rows, gsem):
        pltpu.async_copy(table_hbm.at[idx_v.at[2 * i]],
                         rows.at[pl.ds(0, _HALF)], gsem)
        pltpu.async_copy(table_hbm.at[idx_v.at[2 * i + 1]],
                         rows.at[pl.ds(_HALF, _HALF)], gsem)

    def wait_batch(i, rows, gsem):
        pltpu.make_async_copy(table_hbm.at[idx_v.at[2 * i]],
                              rows.at[pl.ds(0, _HALF)], gsem).wait()
        pltpu.make_async_copy(table_hbm.at[idx_v.at[2 * i + 1]],
                              rows.at[pl.ds(_HALF, _HALF)], gsem).wait()

    for b in range(_NBUF):
        start_batch(b, bufs[b][0], bufs[b][1])

    def body(g, _):
        for b in range(_NBUF):
            i = g * _NBUF + b
            rows, gsem, osem = bufs[b]
            wait_batch(i, rows, gsem)
            pltpu.async_copy(rows, out_hbm.at[bbase + i], osem)

            @pl.when(i + _NBUF < _BPW)
            def _():
                pltpu.make_async_copy(rows, out_hbm.at[bbase + i], osem).wait()
                start_batch(i + _NBUF, rows, gsem)

        return 0

    lax.fori_loop(0, _BPW // _NBUF, body, 0)

    for b in range(_NBUF):
        i = _BPW - _NBUF + b
        rows, _, osem = bufs[b]
        pltpu.make_async_copy(rows, out_hbm.at[bbase + i], osem).wait()


def kernel(inputs, embedding_table):
    idx = jnp.reshape(inputs.astype(jnp.int32), (_K, _NW, 2 * _BPW, _HALF))
    parts = [_gather_kernel(idx[k], embedding_table) for k in range(_K)]
    out = jnp.concatenate(parts, axis=0)
    return (out, embedding_table)


# speedup vs baseline: 1.7127x; 1.6456x over previous
import functools

import jax
import jax.numpy as jnp
from jax import lax
from jax.experimental import pallas as pl
from jax.experimental.pallas import tpu as pltpu
from jax.experimental.pallas import tpu_sc as plsc

VOCAB = 100000
EMBED = 128
BATCH = 1024
SEQ = 200

_info = plsc.get_sparse_core_info()
_NC, _NS = _info.num_cores, _info.num_subcores
_NW = _NC * _NS
_TOTAL = BATCH * SEQ
_PER_W = _TOTAL // _NW
_CL = 128
_NCHUNK = _PER_W // _CL
_NBUF = 5
_GLEAD = 2
_WLAG = _NBUF - _GLEAD


@functools.partial(
    pl.kernel,
    mesh=plsc.VectorSubcoreMesh(core_axis_name="c", subcore_axis_name="s"),
    out_type=jax.ShapeDtypeStruct((_TOTAL, EMBED), jnp.float32),
    scratch_types=(
        [pltpu.VMEM((_NCHUNK, _CL), jnp.int32)]
        + [pltpu.VMEM((_CL, EMBED), jnp.float32)] * _NBUF
        + [pltpu.SemaphoreType.DMA] * (2 * _NBUF)
    ),
)
def _gather_kernel(idx_hbm, table_hbm, out_hbm, idx_v, *bufs_and_sems):
    rows_bufs = bufs_and_sems[:_NBUF]
    gsems = bufs_and_sems[_NBUF:2 * _NBUF]
    osems = bufs_and_sems[2 * _NBUF:]
    wid = lax.axis_index("s") * _NC + lax.axis_index("c")
    base = wid * _PER_W

    pltpu.sync_copy(idx_hbm.at[wid], idx_v)

    def gather(j, b):
        pltpu.async_copy(table_hbm.at[idx_v.at[j]], rows_bufs[b], gsems[b])

    def wait_gather(j, b):
        pltpu.make_async_copy(table_hbm.at[idx_v.at[j]], rows_bufs[b],
                              gsems[b]).wait()

    def write(j, b):
        pltpu.async_copy(rows_bufs[b], out_hbm.at[pl.ds(base + j * _CL, _CL)],
                         osems[b])

    def wait_write(j, b):
        pltpu.make_async_copy(rows_bufs[b],
                              out_hbm.at[pl.ds(base + j * _CL, _CL)],
                              osems[b]).wait()

    for b in range(_GLEAD):
        gather(b, b)

    def body(g, _):
        for b in range(_NBUF):
            i = g * _NBUF + b

            @pl.when(i + _GLEAD < _NCHUNK)
            def _():
                @pl.when(i >= _WLAG)
                def _():
                    wait_write(i - _WLAG, (b + _GLEAD) % _NBUF)
                gather(i + _GLEAD, (b + _GLEAD) % _NBUF)

            wait_gather(i, b)
            write(i, b)

        return 0

    lax.fori_loop(0, _NCHUNK // _NBUF, body, 0)

    for b in range(_NBUF):
        i = _NCHUNK - _NBUF + b
        wait_write(i, i % _NBUF)


def kernel(inputs, embedding_table):
    idx = jnp.reshape(inputs.astype(jnp.int32), (_NW, _NCHUNK, _CL))
    out = _gather_kernel(idx, embedding_table)
    return (jnp.reshape(out, (BATCH, SEQ, EMBED)), embedding_table)


# speedup vs baseline: 1.7480x; 1.0206x over previous
import functools

import jax
import jax.numpy as jnp
from jax import lax
from jax.experimental import pallas as pl
from jax.experimental.pallas import tpu as pltpu
from jax.experimental.pallas import tpu_sc as plsc

VOCAB = 100000
EMBED = 128
BATCH = 1024
SEQ = 200

_info = plsc.get_sparse_core_info()
_NC, _NS = _info.num_cores, _info.num_subcores
_NW = _NC * _NS
_TOTAL = BATCH * SEQ
_PER_W = _TOTAL // _NW
_CL = 64
_NCHUNK = _PER_W // _CL
_NBUF = 10
_GLEAD = 4
_WLAG = _NBUF - _GLEAD


@functools.partial(
    pl.kernel,
    mesh=plsc.VectorSubcoreMesh(core_axis_name="c", subcore_axis_name="s"),
    out_type=jax.ShapeDtypeStruct((_TOTAL, EMBED), jnp.float32),
    scratch_types=(
        [pltpu.VMEM((_NCHUNK, _CL), jnp.int32)]
        + [pltpu.VMEM((_CL, EMBED), jnp.float32)] * _NBUF
        + [pltpu.SemaphoreType.DMA] * (2 * _NBUF)
    ),
)
def _gather_kernel(idx_hbm, table_hbm, out_hbm, idx_v, *bufs_and_sems):
    rows_bufs = bufs_and_sems[:_NBUF]
    gsems = bufs_and_sems[_NBUF:2 * _NBUF]
    osems = bufs_and_sems[2 * _NBUF:]
    wid = lax.axis_index("s") * _NC + lax.axis_index("c")
    base = wid * _PER_W

    pltpu.sync_copy(idx_hbm.at[wid], idx_v)

    def gather(j, b):
        pltpu.async_copy(table_hbm.at[idx_v.at[j]], rows_bufs[b], gsems[b])

    def wait_gather(j, b):
        pltpu.make_async_copy(table_hbm.at[idx_v.at[j]], rows_bufs[b],
                              gsems[b]).wait()

    def write(j, b):
        pltpu.async_copy(rows_bufs[b], out_hbm.at[pl.ds(base + j * _CL, _CL)],
                         osems[b])

    def wait_write(j, b):
        pltpu.make_async_copy(rows_bufs[b],
                              out_hbm.at[pl.ds(base + j * _CL, _CL)],
                              osems[b]).wait()

    for b in range(_GLEAD):
        gather(b, b)

    def body(g, _):
        for b in range(_NBUF):
            i = g * _NBUF + b

            @pl.when(i + _GLEAD < _NCHUNK)
            def _():
                @pl.when(i >= _WLAG)
                def _():
                    wait_write(i - _WLAG, (b + _GLEAD) % _NBUF)
                gather(i + _GLEAD, (b + _GLEAD) % _NBUF)

            wait_gather(i, b)
            write(i, b)

        return 0

    lax.fori_loop(0, _NCHUNK // _NBUF, body, 0)

    for b in range(_NBUF):
        i = _NCHUNK - _NBUF + b
        wait_write(i, i % _NBUF)


_TBLK = 10000


def _copy_body(src_ref, dst_ref):
    dst_ref[...] = src_ref[...]


_table_copy = pl.pallas_call(
    _copy_body,
    out_shape=jax.ShapeDtypeStruct((VOCAB, EMBED), jnp.float32),
    grid=(VOCAB // _TBLK,),
    in_specs=[pl.BlockSpec((_TBLK, EMBED), lambda i: (i, 0))],
    out_specs=pl.BlockSpec((_TBLK, EMBED), lambda i: (i, 0)),
)


def kernel(inputs, embedding_table):
    idx = jnp.reshape(inputs.astype(jnp.int32), (_NW, _NCHUNK, _CL))
    tbl = _table_copy(embedding_table)
    out = _gather_kernel(idx, embedding_table)
    return (jnp.reshape(out, (BATCH, SEQ, EMBED)), tbl)


# speedup vs baseline: 1.7508x; 1.0016x over previous
import functools

import jax
import jax.numpy as jnp
from jax import lax
from jax.experimental import pallas as pl
from jax.experimental.pallas import tpu as pltpu
from jax.experimental.pallas import tpu_sc as plsc

VOCAB = 100000
EMBED = 128
BATCH = 1024
SEQ = 200

_info = plsc.get_sparse_core_info()
_NC, _NS = _info.num_cores, _info.num_subcores
_NW = _NC * _NS
_TOTAL = BATCH * SEQ
_PER_W = _TOTAL // _NW
_CL = 128
_NCHUNK = _PER_W // _CL
_NBUF = 5
_GLEAD = 2
_WLAG = _NBUF - _GLEAD


@functools.partial(
    pl.kernel,
    mesh=plsc.VectorSubcoreMesh(core_axis_name="c", subcore_axis_name="s"),
    out_type=jax.ShapeDtypeStruct((_TOTAL, EMBED), jnp.float32),
    scratch_types=(
        [pltpu.VMEM((_NCHUNK, _CL), jnp.int32)]
        + [pltpu.VMEM((_CL, EMBED), jnp.float32)] * _NBUF
        + [pltpu.SemaphoreType.DMA] * (2 * _NBUF)
    ),
)
def _gather_kernel(idx_hbm, table_hbm, out_hbm, idx_v, *bufs_and_sems):
    rows_bufs = bufs_and_sems[:_NBUF]
    gsems = bufs_and_sems[_NBUF:2 * _NBUF]
    osems = bufs_and_sems[2 * _NBUF:]
    wid = lax.axis_index("s") * _NC + lax.axis_index("c")
    base = wid * _PER_W

    pltpu.sync_copy(idx_hbm.at[wid], idx_v)

    def gather(j, b):
        pltpu.async_copy(table_hbm.at[idx_v.at[j]], rows_bufs[b], gsems[b])

    def wait_gather(j, b):
        pltpu.make_async_copy(table_hbm.at[idx_v.at[j]], rows_bufs[b],
                              gsems[b]).wait()

    def write(j, b):
        pltpu.async_copy(rows_bufs[b], out_hbm.at[pl.ds(base + j * _CL, _CL)],
                         osems[b])

    def wait_write(j, b):
        pltpu.make_async_copy(rows_bufs[b],
                              out_hbm.at[pl.ds(base + j * _CL, _CL)],
                              osems[b]).wait()

    for b in range(_GLEAD):
        gather(b, b)

    def body(g, _):
        for b in range(_NBUF):
            i = g * _NBUF + b

            @pl.when(i + _GLEAD < _NCHUNK)
            def _():
                @pl.when(i >= _WLAG)
                def _():
                    wait_write(i - _WLAG, (b + _GLEAD) % _NBUF)
                gather(i + _GLEAD, (b + _GLEAD) % _NBUF)

            wait_gather(i, b)
            write(i, b)

        return 0

    lax.fori_loop(0, _NCHUNK // _NBUF, body, 0)

    for b in range(_NBUF):
        i = _NCHUNK - _NBUF + b
        wait_write(i, i % _NBUF)


_TBLK = 10000


def _copy_body(src_ref, dst_ref):
    dst_ref[...] = src_ref[...]


_table_copy = pl.pallas_call(
    _copy_body,
    out_shape=jax.ShapeDtypeStruct((VOCAB, EMBED), jnp.float32),
    grid=(VOCAB // _TBLK,),
    in_specs=[pl.BlockSpec((_TBLK, EMBED), lambda i: (i, 0))],
    out_specs=pl.BlockSpec((_TBLK, EMBED), lambda i: (i, 0)),
)


def kernel(inputs, embedding_table):
    idx = jnp.reshape(inputs.astype(jnp.int32), (_NW, _NCHUNK, _CL))
    tbl = _table_copy(embedding_table)
    out = _gather_kernel(idx, embedding_table)
    return (jnp.reshape(out, (BATCH, SEQ, EMBED)), tbl)
